# baseline (speedup 1.0000x reference)
import functools

import jax
import jax.numpy as jnp
from jax import lax
from jax.experimental import pallas as pl
from jax.experimental.pallas import tpu as pltpu
from jax.experimental.pallas import tpu_sc as plsc

N = 10000
D = 128
CLS = 16

NC = 2
NS = 16
NW = NC * NS

NP = 10240
E = 320000
CH = 128
NCHUNK = 80
NBUF = 4
EPT = CH * NCHUNK
EP = EPT * NW
RPW = NP // NS

_MESH = dict(core_axis_name="c", subcore_axis_name="s")


def _fill_const(ref, rows, width, value, dtype=jnp.float32):
    lanes = 32 if dtype == jnp.bfloat16 else 16

    def body(i, _):
        for j in range(width // lanes):
            ref[i, pl.ds(j * lanes, lanes)] = jnp.full((lanes,), value, dtype)
        return 0
    lax.fori_loop(0, rows, body, 0)


def _sc_degrees(srcp2, dstp2):
    @functools.partial(
        pl.kernel,
        out_type=jax.ShapeDtypeStruct((NC, 2, NP, 16), jnp.float32),
        mesh=plsc.VectorSubcoreMesh(**_MESH),
        scratch_types=[
            pltpu.VMEM_SHARED((NP, 16), jnp.float32),
            pltpu.VMEM_SHARED((NP, 16), jnp.float32),
            pltpu.VMEM((NCHUNK, CH), jnp.int32),
            pltpu.VMEM((NCHUNK, CH), jnp.int32),
            pltpu.VMEM((CH, 16), jnp.float32),
            pltpu.VMEM((RPW, 16), jnp.float32),
        ] + [pltpu.SemaphoreType.DMA] * NBUF,
        compiler_params=pltpu.CompilerParams(use_tc_tiling_on_sc=False),
    )
    def deg_kernel(src_hbm, dst_hbm, out_hbm, acc_s, acc_d, sidx, didx,
                   ones_v, zer_v, *sems):
        c = lax.axis_index("c")
        s = lax.axis_index("s")
        wid = c * NS + s
        _fill_const(ones_v, CH, 16, 1.0)
        _fill_const(zer_v, RPW, 16, 0.0)
        pltpu.sync_copy(src_hbm.at[pl.ds(wid * NCHUNK, NCHUNK)], sidx)
        pltpu.sync_copy(dst_hbm.at[pl.ds(wid * NCHUNK, NCHUNK)], didx)
        pltpu.sync_copy(zer_v, acc_s.at[pl.ds(s * RPW, RPW)])
        pltpu.sync_copy(zer_v, acc_d.at[pl.ds(s * RPW, RPW)])
        plsc.subcore_barrier()

        def fire(k, b):
            pltpu.async_copy(ones_v, acc_s.at[sidx.at[k]], sems[b], add=True)
            pltpu.async_copy(ones_v, acc_d.at[didx.at[k]], sems[b], add=True)

        def drain(k, b):
            pltpu.make_async_copy(ones_v, acc_s.at[sidx.at[k]], sems[b]).wait()
            pltpu.make_async_copy(ones_v, acc_d.at[didx.at[k]], sems[b]).wait()

        for b in range(NBUF):
            fire(b, b)

        def step(kk, _):
            for b in range(NBUF):
                k = kk * NBUF + b
                drain(k, b)
                fire(k + NBUF, b)
            return 0

        lax.fori_loop(0, NCHUNK // NBUF - 1, step, 0)
        for b in range(NBUF):
            drain(NCHUNK - NBUF + b, b)
        plsc.subcore_barrier()
        pltpu.sync_copy(acc_s.at[pl.ds(s * RPW, RPW)],
                        out_hbm.at[c, 0, pl.ds(s * RPW, RPW)])
        pltpu.sync_copy(acc_d.at[pl.ds(s * RPW, RPW)],
                        out_hbm.at[c, 1, pl.ds(s * RPW, RPW)])

    return deg_kernel(srcp2, dstp2)


NCH2 = EP // CH // NS
HW = D // 2


def _ring_pipeline(tab_hbm, acc, sidx, didx, rows, sems, nch):
    def g(k, b):
        pltpu.async_copy(tab_hbm.at[sidx.at[k]], rows[b], sems[b])

    def sct(k, b):
        pltpu.async_copy(rows[b], acc.at[didx.at[k]], sems[b], add=True)

    def w(b):
        pltpu.make_async_copy(tab_hbm.at[sidx.at[0]], rows[b], sems[b]).wait()

    g(0, 0); g(1, 1)
    g(2, 2); g(3, 3)
    w(0); w(1)
    sct(0, 0); sct(1, 1)

    def body(rr, _):
        for bank in (0, 1):
            r = 2 * rr + bank
            a = (0, 1) if bank == 0 else (2, 3)
            bb = (2, 3) if bank == 0 else (0, 1)
            w(a[0]); w(a[1])
            g(2 * r, a[0]); g(2 * r + 1, a[1])
            w(bb[0]); w(bb[1])
            sct(2 * (r - 1), bb[0]); sct(2 * (r - 1) + 1, bb[1])
        return 0

    lax.fori_loop(1, nch // 4, body, 0)
    last = nch // 2 - 1
    w(2); w(3)
    sct(2 * last, 2); sct(2 * last + 1, 3)
    w(0); w(1)
    w(2); w(3)


def _sc_scatter_cols(table2, srcp2, dstp2):
    @functools.partial(
        pl.kernel,
        out_type=jax.ShapeDtypeStruct((NC, NP, HW), jnp.bfloat16),
        mesh=plsc.VectorSubcoreMesh(**_MESH),
        scratch_types=[
            pltpu.VMEM_SHARED((NP, HW), jnp.bfloat16),
            pltpu.VMEM((NCH2, CH), jnp.int32),
            pltpu.VMEM((NCH2, CH), jnp.int32),
        ] + [pltpu.VMEM((CH, HW), jnp.bfloat16)] * NBUF + [
            pltpu.VMEM((64, HW), jnp.bfloat16),
        ] + [pltpu.SemaphoreType.DMA] * NBUF,
        compiler_params=pltpu.CompilerParams(use_tc_tiling_on_sc=False),
    )
    def scat_kernel(tab_hbm, src_hbm, dst_hbm, out_hbm,
                    acc, sidx, didx, *rest):
        rows = rest[:NBUF]
        zer_v = rest[NBUF]
        sems = rest[NBUF + 1:]
        c = lax.axis_index("c")
        s = lax.axis_index("s")
        pltpu.sync_copy(src_hbm.at[pl.ds(s * NCH2, NCH2)], sidx)
        pltpu.sync_copy(dst_hbm.at[pl.ds(s * NCH2, NCH2)], didx)

        @pl.when(c == 1)
        def _():
            off = jnp.full((16,), NP, jnp.int32)

            def shift(i, _):
                for j in range(CH // 16):
                    sl = pl.ds(j * 16, 16)
                    sidx[i, sl] = sidx[i, sl] + off
                return 0
            lax.fori_loop(0, NCH2, shift, 0)

        _fill_const(zer_v, 64, HW, 0.0, jnp.bfloat16)
        for r in range(RPW // 64):
            pltpu.sync_copy(zer_v, acc.at[pl.ds(s * RPW + r * 64, 64)])
        plsc.subcore_barrier()
        _ring_pipeline(tab_hbm, acc, sidx, didx, rows, sems, NCH2)
        plsc.subcore_barrier()
        pltpu.sync_copy(acc.at[pl.ds(s * RPW, RPW)],
                        out_hbm.at[c, pl.ds(s * RPW, RPW)])

    return scat_kernel(table2, srcp2, dstp2)


def _sc_scatter16(table, srcp2, dstp2):
    @functools.partial(
        pl.kernel,
        out_type=jax.ShapeDtypeStruct((NC, NP, CLS), jnp.float32),
        mesh=plsc.VectorSubcoreMesh(**_MESH),
        scratch_types=[
            pltpu.VMEM_SHARED((NP, CLS), jnp.float32),
            pltpu.VMEM((NCHUNK, CH), jnp.int32),
            pltpu.VMEM((NCHUNK, CH), jnp.int32),
        ] + [pltpu.VMEM((CH, CLS), jnp.float32)] * NBUF + [
            pltpu.VMEM((RPW, CLS), jnp.float32),
        ] + [pltpu.SemaphoreType.DMA] * NBUF,
        compiler_params=pltpu.CompilerParams(use_tc_tiling_on_sc=False),
    )
    def scat_kernel(tab_hbm, src_hbm, dst_hbm, out_hbm,
                    acc, sidx, didx, *rest):
        rows = rest[:NBUF]
        zer_v = rest[NBUF]
        sems = rest[NBUF + 1:]
        c = lax.axis_index("c")
        s = lax.axis_index("s")
        wid = c * NS + s
        pltpu.sync_copy(src_hbm.at[pl.ds(wid * NCHUNK, NCHUNK)], sidx)
        pltpu.sync_copy(dst_hbm.at[pl.ds(wid * NCHUNK, NCHUNK)], didx)
        _fill_const(zer_v, RPW, CLS, 0.0)
        pltpu.sync_copy(zer_v, acc.at[pl.ds(s * RPW, RPW)])
        plsc.subcore_barrier()
        _ring_pipeline(tab_hbm, acc, sidx, didx, rows, sems, NCHUNK)
        plsc.subcore_barrier()
        pltpu.sync_copy(acc.at[pl.ds(s * RPW, RPW)],
                        out_hbm.at[c, pl.ds(s * RPW, RPW)])

    return scat_kernel(table, srcp2, dstp2)


BLK = 512
BLK3 = 400


def _deg_spec():
    return pl.BlockSpec((NC, 2, BLK, 16), lambda i: (0, 0, i, 0))


def _norm(deg_ref, which):
    d = deg_ref[0, which, :, 0:1] + deg_ref[1, which, :, 0:1]
    return lax.rsqrt(jnp.maximum(d, 1.0))


def _tc_m1(xp, W1, degs):
    def body(x_ref, w_ref, deg_ref, o_ref):
        i = pl.program_id(1)
        y = jnp.dot(x_ref[...], w_ref[0], preferred_element_type=jnp.float32)
        y = y * _norm(deg_ref, 0)
        rowid = i * BLK + lax.broadcasted_iota(jnp.int32, (BLK, 1), 0)
        o_ref[0] = jnp.where(rowid < N, y, 0.0).astype(jnp.bfloat16)

    return pl.pallas_call(
        body,
        grid=(2, NP // BLK),
        in_specs=[
            pl.BlockSpec((BLK, D), lambda h, i: (i, 0)),
            pl.BlockSpec((1, D, HW), lambda h, i: (h, 0, 0)),
            pl.BlockSpec((NC, 2, BLK, 16), lambda h, i: (0, 0, i, 0)),
        ],
        out_specs=pl.BlockSpec((1, BLK, HW), lambda h, i: (h, i, 0)),
        out_shape=jax.ShapeDtypeStruct((2, NP, HW), jnp.bfloat16),
    )(xp, jnp.stack([W1[:, :HW], W1[:, HW:]]), degs)


def _tc_m2(p, degs, b1, W2):
    def body(p_ref, deg_ref, b1_ref, w2_ref, o_ref):
        i = pl.program_id(0)
        agg = jnp.concatenate([p_ref[0], p_ref[1]], axis=-1).astype(jnp.float32)
        h = jnp.maximum(agg * _norm(deg_ref, 1) + b1_ref[...], 0.0)
        h = h * _norm(deg_ref, 0)
        m2 = jnp.dot(h, w2_ref[...], preferred_element_type=jnp.float32)
        rowid = i * BLK + lax.broadcasted_iota(jnp.int32, (BLK, 1), 0)
        o_ref[...] = jnp.where(rowid < N, m2, 0.0)

    return pl.pallas_call(
        body,
        grid=(NP // BLK,),
        in_specs=[
            pl.BlockSpec((NC, BLK, HW), lambda i: (0, i, 0)),
            _deg_spec(),
            pl.BlockSpec((1, D), lambda i: (0, 0)),
            pl.BlockSpec((D, CLS), lambda i: (0, 0)),
        ],
        out_specs=pl.BlockSpec((BLK, CLS), lambda i: (i, 0)),
        out_shape=jax.ShapeDtypeStruct((NP, CLS), jnp.float32),
    )(p, degs, b1, W2)


def _tc_out(q, degs, b2):
    def body(q_ref, deg_ref, b2_ref, o_ref):
        agg = q_ref[0] + q_ref[1]
        o = agg * _norm(deg_ref, 1) + b2_ref[...]
        m = jnp.max(o, axis=-1, keepdims=True)
        e = jnp.exp(o - m)
        o_ref[...] = e / jnp.sum(e, axis=-1, keepdims=True)

    return pl.pallas_call(
        body,
        grid=(N // BLK3,),
        in_specs=[
            pl.BlockSpec((NC, BLK3, CLS), lambda i: (0, i, 0)),
            pl.BlockSpec((NC, 2, BLK3, 16), lambda i: (0, 0, i, 0)),
            pl.BlockSpec((1, CLS), lambda i: (0, 0)),
        ],
        out_specs=pl.BlockSpec((BLK3, CLS), lambda i: (i, 0)),
        out_shape=jax.ShapeDtypeStruct((N, CLS), jnp.float32),
    )(q, degs, b2)


def kernel(x, edge_index, W1, b1, W2, b2):
    pad = jnp.full((EP - E,), N, jnp.int32)
    srcp = jnp.reshape(jnp.concatenate([edge_index[0], pad]), (EP // CH, CH))
    dstp = jnp.reshape(jnp.concatenate([edge_index[1], pad]), (EP // CH, CH))

    degs = _sc_degrees(srcp, dstp)
    m1 = _tc_m1(x, W1, degs)
    p = _sc_scatter_cols(jnp.reshape(m1, (2 * NP, HW)), srcp, dstp)
    m2 = _tc_m2(p, degs, jnp.reshape(b1, (1, D)), W2)
    q = _sc_scatter16(m2, srcp, dstp)
    return _tc_out(q, degs, jnp.reshape(b2, (1, CLS)))


# speedup vs baseline: 1.0661x; 1.0661x over previous
import functools

import jax
import jax.numpy as jnp
from jax import lax
from jax.experimental import pallas as pl
from jax.experimental.pallas import tpu as pltpu
from jax.experimental.pallas import tpu_sc as plsc

N = 10000
D = 128
CLS = 16

NC = 2
NS = 16
NW = NC * NS

NP = 10240
E = 320000
CH = 128
NCHUNK = 80
NBUF = 4
EPT = CH * NCHUNK
EP = EPT * NW
RPW = NP // NS

_MESH = dict(core_axis_name="c", subcore_axis_name="s")


def _fill_const(ref, rows, width, value, dtype=jnp.float32):
    lanes = 32 if dtype == jnp.bfloat16 else 16

    def body(i, _):
        for j in range(width // lanes):
            ref[i, pl.ds(j * lanes, lanes)] = jnp.full((lanes,), value, dtype)
        return 0
    lax.fori_loop(0, rows, body, 0)


def _sc_degrees(srcp2, dstp2):
    @functools.partial(
        pl.kernel,
        out_type=jax.ShapeDtypeStruct((NC, 2, NP, 16), jnp.float32),
        mesh=plsc.VectorSubcoreMesh(**_MESH),
        scratch_types=[
            pltpu.VMEM_SHARED((NP, 16), jnp.float32),
            pltpu.VMEM_SHARED((NP, 16), jnp.float32),
            pltpu.VMEM((NCHUNK, CH), jnp.int32),
            pltpu.VMEM((NCHUNK, CH), jnp.int32),
            pltpu.VMEM((CH, 16), jnp.float32),
            pltpu.VMEM((RPW, 16), jnp.float32),
        ] + [pltpu.SemaphoreType.DMA] * NBUF,
        compiler_params=pltpu.CompilerParams(use_tc_tiling_on_sc=False),
    )
    def deg_kernel(src_hbm, dst_hbm, out_hbm, acc_s, acc_d, sidx, didx,
                   ones_v, zer_v, *sems):
        c = lax.axis_index("c")
        s = lax.axis_index("s")
        wid = c * NS + s
        _fill_const(ones_v, CH, 16, 1.0)
        _fill_const(zer_v, RPW, 16, 0.0)
        pltpu.sync_copy(src_hbm.at[pl.ds(wid * NCHUNK, NCHUNK)], sidx)
        pltpu.sync_copy(dst_hbm.at[pl.ds(wid * NCHUNK, NCHUNK)], didx)
        pltpu.sync_copy(zer_v, acc_s.at[pl.ds(s * RPW, RPW)])
        pltpu.sync_copy(zer_v, acc_d.at[pl.ds(s * RPW, RPW)])
        plsc.subcore_barrier()

        def fire(k, b):
            pltpu.async_copy(ones_v, acc_s.at[sidx.at[k]], sems[b], add=True)
            pltpu.async_copy(ones_v, acc_d.at[didx.at[k]], sems[b], add=True)

        def drain(k, b):
            pltpu.make_async_copy(ones_v, acc_s.at[sidx.at[k]], sems[b]).wait()
            pltpu.make_async_copy(ones_v, acc_d.at[didx.at[k]], sems[b]).wait()

        for b in range(NBUF):
            fire(b, b)

        def step(kk, _):
            for b in range(NBUF):
                k = kk * NBUF + b
                drain(k, b)
                fire(k + NBUF, b)
            return 0

        lax.fori_loop(0, NCHUNK // NBUF - 1, step, 0)
        for b in range(NBUF):
            drain(NCHUNK - NBUF + b, b)
        plsc.subcore_barrier()
        pltpu.sync_copy(acc_s.at[pl.ds(s * RPW, RPW)],
                        out_hbm.at[c, 0, pl.ds(s * RPW, RPW)])
        pltpu.sync_copy(acc_d.at[pl.ds(s * RPW, RPW)],
                        out_hbm.at[c, 1, pl.ds(s * RPW, RPW)])

    return deg_kernel(srcp2, dstp2)


NCH2 = EP // CH // NS
HW = D // 2


def _ring_pipeline(tab_hbm, acc, sidx, didx, rows, sems, nch):
    def g(k, b):
        pltpu.async_copy(tab_hbm.at[sidx.at[k]], rows[b], sems[b])

    def sct(k, b):
        pltpu.async_copy(rows[b], acc.at[didx.at[k]], sems[b], add=True)

    def w(b):
        pltpu.make_async_copy(tab_hbm.at[sidx.at[0]], rows[b], sems[b]).wait()

    g(0, 0); g(1, 1)
    g(2, 2); g(3, 3)
    w(0); w(1)
    sct(0, 0); sct(1, 1)

    def body(rr, _):
        for bank in (0, 1):
            r = 2 * rr + bank
            a = (0, 1) if bank == 0 else (2, 3)
            bb = (2, 3) if bank == 0 else (0, 1)
            w(a[0]); w(a[1])
            g(2 * r, a[0]); g(2 * r + 1, a[1])
            w(bb[0]); w(bb[1])
            sct(2 * (r - 1), bb[0]); sct(2 * (r - 1) + 1, bb[1])
        return 0

    lax.fori_loop(1, nch // 4, body, 0)
    last = nch // 2 - 1
    w(2); w(3)
    sct(2 * last, 2); sct(2 * last + 1, 3)
    w(0); w(1)
    w(2); w(3)


def _sc_scatter_cols(table2, srcb, dstp2):
    @functools.partial(
        pl.kernel,
        out_type=jax.ShapeDtypeStruct((NC, NP, HW), jnp.bfloat16),
        mesh=plsc.VectorSubcoreMesh(**_MESH),
        scratch_types=[
            pltpu.VMEM_SHARED((NP, HW), jnp.bfloat16),
            pltpu.VMEM((NCH2, CH), jnp.int32),
            pltpu.VMEM((NCH2, CH), jnp.int32),
        ] + [pltpu.VMEM((CH, HW), jnp.bfloat16)] * NBUF + [
            pltpu.VMEM((64, HW), jnp.bfloat16),
        ] + [pltpu.SemaphoreType.DMA] * NBUF,
        compiler_params=pltpu.CompilerParams(use_tc_tiling_on_sc=False),
    )
    def scat_kernel(tab_hbm, src_hbm, dst_hbm, out_hbm,
                    acc, sidx, didx, *rest):
        rows = rest[:NBUF]
        zer_v = rest[NBUF]
        sems = rest[NBUF + 1:]
        c = lax.axis_index("c")
        s = lax.axis_index("s")
        pltpu.sync_copy(src_hbm.at[c, pl.ds(s * NCH2, NCH2)], sidx)
        pltpu.sync_copy(dst_hbm.at[pl.ds(s * NCH2, NCH2)], didx)
        _fill_const(zer_v, 64, HW, 0.0, jnp.bfloat16)
        for r in range(RPW // 64):
            pltpu.sync_copy(zer_v, acc.at[pl.ds(s * RPW + r * 64, 64)])
        plsc.subcore_barrier()
        _ring_pipeline(tab_hbm, acc, sidx, didx, rows, sems, NCH2)
        plsc.subcore_barrier()
        pltpu.sync_copy(acc.at[pl.ds(s * RPW, RPW)],
                        out_hbm.at[c, pl.ds(s * RPW, RPW)])

    return scat_kernel(table2, srcb, dstp2)


def _sc_scatter16(table, srcp2, dstp2):
    @functools.partial(
        pl.kernel,
        out_type=jax.ShapeDtypeStruct((NC, NP, CLS), jnp.float32),
        mesh=plsc.VectorSubcoreMesh(**_MESH),
        scratch_types=[
            pltpu.VMEM_SHARED((NP, CLS), jnp.float32),
            pltpu.VMEM((NCHUNK, CH), jnp.int32),
            pltpu.VMEM((NCHUNK, CH), jnp.int32),
        ] + [pltpu.VMEM((CH, CLS), jnp.float32)] * NBUF + [
            pltpu.VMEM((RPW, CLS), jnp.float32),
        ] + [pltpu.SemaphoreType.DMA] * NBUF,
        compiler_params=pltpu.CompilerParams(use_tc_tiling_on_sc=False),
    )
    def scat_kernel(tab_hbm, src_hbm, dst_hbm, out_hbm,
                    acc, sidx, didx, *rest):
        rows = rest[:NBUF]
        zer_v = rest[NBUF]
        sems = rest[NBUF + 1:]
        c = lax.axis_index("c")
        s = lax.axis_index("s")
        wid = c * NS + s
        pltpu.sync_copy(src_hbm.at[pl.ds(wid * NCHUNK, NCHUNK)], sidx)
        pltpu.sync_copy(dst_hbm.at[pl.ds(wid * NCHUNK, NCHUNK)], didx)
        _fill_const(zer_v, RPW, CLS, 0.0)
        pltpu.sync_copy(zer_v, acc.at[pl.ds(s * RPW, RPW)])
        plsc.subcore_barrier()
        _ring_pipeline(tab_hbm, acc, sidx, didx, rows, sems, NCHUNK)
        plsc.subcore_barrier()
        pltpu.sync_copy(acc.at[pl.ds(s * RPW, RPW)],
                        out_hbm.at[c, pl.ds(s * RPW, RPW)])

    return scat_kernel(table, srcp2, dstp2)


BLK = 512
BLK3 = 400


def _deg_spec():
    return pl.BlockSpec((NC, 2, BLK, 16), lambda i: (0, 0, i, 0))


def _norm(deg_ref, which):
    d = deg_ref[0, which, :, 0:1] + deg_ref[1, which, :, 0:1]
    return lax.rsqrt(jnp.maximum(d, 1.0))


def _tc_m1(xp, W1, degs):
    def body(x_ref, w_ref, deg_ref, o_ref):
        i = pl.program_id(1)
        y = jnp.dot(x_ref[...], w_ref[0], preferred_element_type=jnp.float32)
        y = y * _norm(deg_ref, 0)
        rowid = i * BLK + lax.broadcasted_iota(jnp.int32, (BLK, 1), 0)
        o_ref[0] = jnp.where(rowid < N, y, 0.0).astype(jnp.bfloat16)

    return pl.pallas_call(
        body,
        grid=(2, NP // BLK),
        in_specs=[
            pl.BlockSpec((BLK, D), lambda h, i: (i, 0)),
            pl.BlockSpec((1, D, HW), lambda h, i: (h, 0, 0)),
            pl.BlockSpec((NC, 2, BLK, 16), lambda h, i: (0, 0, i, 0)),
        ],
        out_specs=pl.BlockSpec((1, BLK, HW), lambda h, i: (h, i, 0)),
        out_shape=jax.ShapeDtypeStruct((2, NP, HW), jnp.bfloat16),
    )(xp, jnp.stack([W1[:, :HW], W1[:, HW:]]), degs)


def _tc_m2(p, degs, b1, W2):
    def body(p_ref, deg_ref, b1_ref, w2_ref, o_ref):
        i = pl.program_id(0)
        agg = jnp.concatenate([p_ref[0], p_ref[1]], axis=-1).astype(jnp.float32)
        h = jnp.maximum(agg * _norm(deg_ref, 1) + b1_ref[...], 0.0)
        h = h * _norm(deg_ref, 0)
        m2 = jnp.dot(h, w2_ref[...], preferred_element_type=jnp.float32)
        rowid = i * BLK + lax.broadcasted_iota(jnp.int32, (BLK, 1), 0)
        o_ref[...] = jnp.where(rowid < N, m2, 0.0)

    return pl.pallas_call(
        body,
        grid=(NP // BLK,),
        in_specs=[
            pl.BlockSpec((NC, BLK, HW), lambda i: (0, i, 0)),
            _deg_spec(),
            pl.BlockSpec((1, D), lambda i: (0, 0)),
            pl.BlockSpec((D, CLS), lambda i: (0, 0)),
        ],
        out_specs=pl.BlockSpec((BLK, CLS), lambda i: (i, 0)),
        out_shape=jax.ShapeDtypeStruct((NP, CLS), jnp.float32),
    )(p, degs, b1, W2)


def _tc_out(q, degs, b2):
    def body(q_ref, deg_ref, b2_ref, o_ref):
        agg = q_ref[0] + q_ref[1]
        o = agg * _norm(deg_ref, 1) + b2_ref[...]
        m = jnp.max(o, axis=-1, keepdims=True)
        e = jnp.exp(o - m)
        o_ref[...] = e / jnp.sum(e, axis=-1, keepdims=True)

    return pl.pallas_call(
        body,
        grid=(N // BLK3,),
        in_specs=[
            pl.BlockSpec((NC, BLK3, CLS), lambda i: (0, i, 0)),
            pl.BlockSpec((NC, 2, BLK3, 16), lambda i: (0, 0, i, 0)),
            pl.BlockSpec((1, CLS), lambda i: (0, 0)),
        ],
        out_specs=pl.BlockSpec((BLK3, CLS), lambda i: (i, 0)),
        out_shape=jax.ShapeDtypeStruct((N, CLS), jnp.float32),
    )(q, degs, b2)


def kernel(x, edge_index, W1, b1, W2, b2):
    pad = jnp.full((EP - E,), N, jnp.int32)
    srcp = jnp.reshape(jnp.concatenate([edge_index[0], pad]), (EP // CH, CH))
    dstp = jnp.reshape(jnp.concatenate([edge_index[1], pad]), (EP // CH, CH))

    srcb = jnp.stack([srcp, srcp + NP])
    degs = _sc_degrees(srcp, dstp)
    m1 = _tc_m1(x, W1, degs)
    p = _sc_scatter_cols(jnp.reshape(m1, (2 * NP, HW)), srcb, dstp)
    m2 = _tc_m2(p, degs, jnp.reshape(b1, (1, D)), W2)
    q = _sc_scatter16(m2, srcp, dstp)
    return _tc_out(q, degs, jnp.reshape(b2, (1, CLS)))


# speedup vs baseline: 1.2097x; 1.1347x over previous
import functools

import jax
import jax.numpy as jnp
from jax import lax
from jax.experimental import pallas as pl
from jax.experimental.pallas import tpu as pltpu
from jax.experimental.pallas import tpu_sc as plsc

N = 10000
D = 128
CLS = 16

NC = 2
NS = 16
NW = NC * NS

NP = 10240
E = 320000
CH = 128
NCHUNK = 80
NBUF = 4
EPT = CH * NCHUNK
EP = EPT * NW
RPW = NP // NS

_MESH = dict(core_axis_name="c", subcore_axis_name="s")


def _fill_const(ref, rows, width, value, dtype=jnp.float32):
    lanes = 32 if dtype == jnp.bfloat16 else 16

    def body(i, _):
        for j in range(width // lanes):
            ref[i, pl.ds(j * lanes, lanes)] = jnp.full((lanes,), value, dtype)
        return 0
    lax.fori_loop(0, rows, body, 0)


def _sc_degrees(srcp2, dstp2):
    @functools.partial(
        pl.kernel,
        out_type=jax.ShapeDtypeStruct((NC, 2, NP, 16), jnp.float32),
        mesh=plsc.VectorSubcoreMesh(**_MESH),
        scratch_types=[
            pltpu.VMEM_SHARED((NP, 16), jnp.float32),
            pltpu.VMEM_SHARED((NP, 16), jnp.float32),
            pltpu.VMEM((NCHUNK, CH), jnp.int32),
            pltpu.VMEM((NCHUNK, CH), jnp.int32),
            pltpu.VMEM((CH, 16), jnp.float32),
            pltpu.VMEM((RPW, 16), jnp.float32),
        ] + [pltpu.SemaphoreType.DMA] * NBUF,
        compiler_params=pltpu.CompilerParams(use_tc_tiling_on_sc=False),
    )
    def deg_kernel(src_hbm, dst_hbm, out_hbm, acc_s, acc_d, sidx, didx,
                   ones_v, zer_v, *sems):
        c = lax.axis_index("c")
        s = lax.axis_index("s")
        wid = c * NS + s
        _fill_const(ones_v, CH, 16, 1.0)
        _fill_const(zer_v, RPW, 16, 0.0)
        pltpu.sync_copy(src_hbm.at[pl.ds(wid * NCHUNK, NCHUNK)], sidx)
        pltpu.sync_copy(dst_hbm.at[pl.ds(wid * NCHUNK, NCHUNK)], didx)
        pltpu.sync_copy(zer_v, acc_s.at[pl.ds(s * RPW, RPW)])
        pltpu.sync_copy(zer_v, acc_d.at[pl.ds(s * RPW, RPW)])
        plsc.subcore_barrier()

        def fire(k, b):
            pltpu.async_copy(ones_v, acc_s.at[sidx.at[k]], sems[b], add=True)
            pltpu.async_copy(ones_v, acc_d.at[didx.at[k]], sems[b], add=True)

        def drain(k, b):
            pltpu.make_async_copy(ones_v, acc_s.at[sidx.at[k]], sems[b]).wait()
            pltpu.make_async_copy(ones_v, acc_d.at[didx.at[k]], sems[b]).wait()

        for b in range(NBUF):
            fire(b, b)

        def step(kk, _):
            for b in range(NBUF):
                k = kk * NBUF + b
                drain(k, b)
                fire(k + NBUF, b)
            return 0

        lax.fori_loop(0, NCHUNK // NBUF - 1, step, 0)
        for b in range(NBUF):
            drain(NCHUNK - NBUF + b, b)
        plsc.subcore_barrier()
        pltpu.sync_copy(acc_s.at[pl.ds(s * RPW, RPW)],
                        out_hbm.at[c, 0, pl.ds(s * RPW, RPW)])
        pltpu.sync_copy(acc_d.at[pl.ds(s * RPW, RPW)],
                        out_hbm.at[c, 1, pl.ds(s * RPW, RPW)])

    return deg_kernel(srcp2, dstp2)


NCH2 = EP // CH // NS
HW = D // 2


def _ring_pipeline(tab_hbm, acc, sidx, didx, rows, sems, nch):
    def g(k, b):
        pltpu.async_copy(tab_hbm.at[sidx.at[k]], rows[b], sems[b])

    def sct(k, b):
        pltpu.async_copy(rows[b], acc.at[didx.at[k]], sems[b], add=True)

    def w(b):
        pltpu.make_async_copy(tab_hbm.at[sidx.at[0]], rows[b], sems[b]).wait()

    g(0, 0); g(1, 1)
    g(2, 2); g(3, 3)
    w(0); w(1)
    sct(0, 0); sct(1, 1)

    def body(rr, _):
        for bank in (0, 1):
            r = 2 * rr + bank
            a = (0, 1) if bank == 0 else (2, 3)
            bb = (2, 3) if bank == 0 else (0, 1)
            w(a[0]); w(a[1])
            g(2 * r, a[0]); g(2 * r + 1, a[1])
            w(bb[0]); w(bb[1])
            sct(2 * (r - 1), bb[0]); sct(2 * (r - 1) + 1, bb[1])
        return 0

    lax.fori_loop(1, nch // 4, body, 0)
    last = nch // 2 - 1
    w(2); w(3)
    sct(2 * last, 2); sct(2 * last + 1, 3)
    w(0); w(1)
    w(2); w(3)


def _sc_scatter_cols(table2, srcb, dstp2):
    @functools.partial(
        pl.kernel,
        out_type=jax.ShapeDtypeStruct((NC, NP, HW), jnp.bfloat16),
        mesh=plsc.VectorSubcoreMesh(**_MESH),
        scratch_types=[
            pltpu.VMEM_SHARED((NP, HW), jnp.bfloat16),
            pltpu.VMEM((NCH2, CH), jnp.int32),
            pltpu.VMEM((NCH2, CH), jnp.int32),
        ] + [pltpu.VMEM((CH, HW), jnp.bfloat16)] * NBUF + [
            pltpu.VMEM((64, HW), jnp.bfloat16),
        ] + [pltpu.SemaphoreType.DMA] * NBUF,
        compiler_params=pltpu.CompilerParams(use_tc_tiling_on_sc=False),
    )
    def scat_kernel(tab_hbm, src_hbm, dst_hbm, out_hbm,
                    acc, sidx, didx, *rest):
        rows = rest[:NBUF]
        zer_v = rest[NBUF]
        sems = rest[NBUF + 1:]
        c = lax.axis_index("c")
        s = lax.axis_index("s")
        pltpu.sync_copy(src_hbm.at[c, pl.ds(s * NCH2, NCH2)], sidx)
        pltpu.sync_copy(dst_hbm.at[pl.ds(s * NCH2, NCH2)], didx)
        _fill_const(zer_v, 64, HW, 0.0, jnp.bfloat16)
        for r in range(RPW // 64):
            pltpu.sync_copy(zer_v, acc.at[pl.ds(s * RPW + r * 64, 64)])
        plsc.subcore_barrier()
        _ring_pipeline(tab_hbm, acc, sidx, didx, rows, sems, NCH2)
        plsc.subcore_barrier()
        pltpu.sync_copy(acc.at[pl.ds(s * RPW, RPW)],
                        out_hbm.at[c, pl.ds(s * RPW, RPW)])

    return scat_kernel(table2, srcb, dstp2)


def _sc_scatter16(table, srcp2, dstp2):
    @functools.partial(
        pl.kernel,
        out_type=jax.ShapeDtypeStruct((NC, NP, CLS), jnp.float32),
        mesh=plsc.VectorSubcoreMesh(**_MESH),
        scratch_types=[
            pltpu.VMEM_SHARED((NP, CLS), jnp.float32),
            pltpu.VMEM((NCHUNK, CH), jnp.int32),
            pltpu.VMEM((NCHUNK, CH), jnp.int32),
        ] + [pltpu.VMEM((CH, CLS), jnp.float32)] * NBUF + [
            pltpu.VMEM((RPW, CLS), jnp.float32),
        ] + [pltpu.SemaphoreType.DMA] * NBUF,
        compiler_params=pltpu.CompilerParams(use_tc_tiling_on_sc=False),
    )
    def scat_kernel(tab_hbm, src_hbm, dst_hbm, out_hbm,
                    acc, sidx, didx, *rest):
        rows = rest[:NBUF]
        zer_v = rest[NBUF]
        sems = rest[NBUF + 1:]
        c = lax.axis_index("c")
        s = lax.axis_index("s")
        wid = c * NS + s
        pltpu.sync_copy(src_hbm.at[pl.ds(wid * NCHUNK, NCHUNK)], sidx)
        pltpu.sync_copy(dst_hbm.at[pl.ds(wid * NCHUNK, NCHUNK)], didx)
        _fill_const(zer_v, RPW, CLS, 0.0)
        pltpu.sync_copy(zer_v, acc.at[pl.ds(s * RPW, RPW)])
        plsc.subcore_barrier()
        _ring_pipeline(tab_hbm, acc, sidx, didx, rows, sems, NCHUNK)
        plsc.subcore_barrier()
        pltpu.sync_copy(acc.at[pl.ds(s * RPW, RPW)],
                        out_hbm.at[c, pl.ds(s * RPW, RPW)])

    return scat_kernel(table, srcp2, dstp2)


def _norm(deg_ref, which):
    d = deg_ref[0, which, :, 0:1] + deg_ref[1, which, :, 0:1]
    return lax.rsqrt(jnp.maximum(d, 1.0))


def _tc_m1(x, W1, degs):
    def body(x_ref, w_ref, deg_ref, o_ref):
        y = jnp.dot(x_ref[...], w_ref[...], preferred_element_type=jnp.float32)
        d = deg_ref[0, 0, :, 0:1] + deg_ref[1, 0, :, 0:1]
        ns = lax.rsqrt(jnp.maximum(d, 1.0))[:N]
        yb = (y * ns).astype(jnp.bfloat16)
        zpad = jnp.zeros((NP - N, HW), jnp.bfloat16)
        o_ref[pl.ds(0, N)] = yb[:, :HW]
        o_ref[pl.ds(N, NP - N)] = zpad
        o_ref[pl.ds(NP, N)] = yb[:, HW:]
        o_ref[pl.ds(NP + N, NP - N)] = zpad
    return pl.pallas_call(
        body,
        out_shape=jax.ShapeDtypeStruct((2 * NP, HW), jnp.bfloat16),
    )(x, W1, degs)


def _tc_m2(p, degs, b1, W2):
    def body(p_ref, deg_ref, b1_ref, w2_ref, o_ref):
        agg = jnp.concatenate([p_ref[0], p_ref[1]], axis=-1).astype(jnp.float32)
        h = jnp.maximum(agg * _norm(deg_ref, 1) + b1_ref[...], 0.0)
        h = h * _norm(deg_ref, 0)
        m2 = jnp.dot(h, w2_ref[...], preferred_element_type=jnp.float32)
        rowid = lax.broadcasted_iota(jnp.int32, (NP, 1), 0)
        o_ref[...] = jnp.where(rowid < N, m2, 0.0)
    return pl.pallas_call(
        body,
        out_shape=jax.ShapeDtypeStruct((NP, CLS), jnp.float32),
    )(p, degs, b1, W2)


def _tc_out(q, degs, b2):
    def body(q_ref, deg_ref, b2_ref, o_ref):
        agg = q_ref[0, :N] + q_ref[1, :N]
        d = deg_ref[0, 1, :, 0:1] + deg_ref[1, 1, :, 0:1]
        nd = lax.rsqrt(jnp.maximum(d, 1.0))[:N]
        o = agg * nd + b2_ref[...]
        m = jnp.max(o, axis=-1, keepdims=True)
        e = jnp.exp(o - m)
        o_ref[...] = e / jnp.sum(e, axis=-1, keepdims=True)
    return pl.pallas_call(
        body,
        out_shape=jax.ShapeDtypeStruct((N, CLS), jnp.float32),
    )(q, degs, b2)


def kernel(x, edge_index, W1, b1, W2, b2):
    pad = jnp.full((EP - E,), N, jnp.int32)
    srcp = jnp.reshape(jnp.concatenate([edge_index[0], pad]), (EP // CH, CH))
    dstp = jnp.reshape(jnp.concatenate([edge_index[1], pad]), (EP // CH, CH))

    srcb = jnp.stack([srcp, srcp + NP])
    degs = _sc_degrees(srcp, dstp)
    m1 = _tc_m1(x, W1, degs)
    p = _sc_scatter_cols(m1, srcb, dstp)
    m2 = _tc_m2(p, degs, jnp.reshape(b1, (1, D)), W2)
    q = _sc_scatter16(m2, srcp, dstp)
    return _tc_out(q, degs, jnp.reshape(b2, (1, CLS)))
